# Initial kernel scaffold; baseline (speedup 1.0000x reference)
import jax
import jax.numpy as jnp
from jax.experimental import pallas as pl


def kernel(x, table, W, b):
    raise NotImplementedError("write your pallas kernel here")


# speedup vs baseline: 7.0759x; 7.0759x over previous
import functools

import jax
import jax.numpy as jnp
from jax import lax
from jax.experimental import pallas as pl
from jax.experimental.pallas import tpu as pltpu
from jax.experimental.pallas import tpu_sc as plsc

_NUM_UNIQ = [1000000, 1000000]
_D = 32
_F = 2

_NC = 2
_NS = 16
_NW = _NC * _NS
_LANES = 16

_SUB = 128


def _sc_gather(x_flat, table, rows_total, chunk, field1_off):
    per_w = rows_total // _NW
    n_chunks = per_w // chunk
    n_sub = chunk // _SUB
    n_vec = chunk // _LANES

    mesh = plsc.VectorSubcoreMesh(core_axis_name="c", subcore_axis_name="s")

    @functools.partial(
        pl.kernel,
        mesh=mesh,
        out_type=jax.ShapeDtypeStruct((rows_total, _D), jnp.float32),
        scratch_types=[
            pltpu.VMEM((chunk,), jnp.int32),
            pltpu.VMEM((chunk, _D), jnp.float32),
            pltpu.SemaphoreType.DMA,
        ],
        compiler_params=pltpu.CompilerParams(use_tc_tiling_on_sc=False),
    )
    def gather_kernel(table_hbm, idx_hbm, out_hbm, idx_v, rows_v, sem):
        wid = lax.axis_index("s") * _NC + lax.axis_index("c")
        base = wid * per_w
        pat = (lax.iota(jnp.int32, 16) & 1) * field1_off

        def chunk_body(i, carry):
            off = pl.multiple_of(base + i * chunk, _SUB)
            pltpu.sync_copy(idx_hbm.at[pl.ds(off, chunk)], idx_v)
            for j in range(n_vec):
                idx_v[pl.ds(j * _LANES, _LANES)] = (
                    idx_v[pl.ds(j * _LANES, _LANES)] + pat
                )
            handles = []
            for j in range(n_sub):
                handles.append(
                    pltpu.async_copy(
                        table_hbm.at[idx_v.at[pl.ds(j * _SUB, _SUB)]],
                        rows_v.at[pl.ds(j * _SUB, _SUB)],
                        sem,
                    )
                )
            for h in handles:
                h.wait()
            pltpu.sync_copy(rows_v, out_hbm.at[pl.ds(off, chunk)])
            return carry

        lax.fori_loop(0, n_chunks, chunk_body, 0)

    return gather_kernel(table, x_flat)


def _tc_project(emb, wt, b2d, tile_n):
    n, fd = emb.shape
    d = wt.shape[1]

    def mm_kernel(emb_ref, wt_ref, b_ref, out_ref):
        out_ref[...] = (
            jnp.dot(emb_ref[...], wt_ref[...],
                    preferred_element_type=jnp.float32)
            + b_ref[...]
        )

    return pl.pallas_call(
        mm_kernel,
        grid=(n // tile_n,),
        in_specs=[
            pl.BlockSpec((tile_n, fd), lambda i: (i, 0)),
            pl.BlockSpec((fd, d), lambda i: (0, 0)),
            pl.BlockSpec((1, d), lambda i: (0, 0)),
        ],
        out_specs=pl.BlockSpec((tile_n, d), lambda i: (i, 0)),
        out_shape=jax.ShapeDtypeStruct((n, d), jnp.float32),
    )(emb, wt, b2d)


def kernel(x, table, W, b):
    B, S, G, F = x.shape
    n_rows = B * S * G
    rows_total = n_rows * F

    x_flat = x.reshape(rows_total)
    gathered = _sc_gather(x_flat, table, rows_total, chunk=1280,
                          field1_off=_NUM_UNIQ[0])
    emb = gathered.reshape(n_rows, F * _D)
    out = _tc_project(emb, W.T, b.reshape(1, _D), tile_n=2048)
    return out.reshape(B, S, G, _D)
